# baseline (speedup 1.0000x reference)
import functools

import jax
import jax.numpy as jnp
import numpy as np
from jax import lax
from jax.experimental import pallas as pl
from jax.experimental.pallas import tpu as pltpu
from jax.experimental.pallas import tpu_sc as plsc

_VOCAB = 1000
_DIM = 128
_N_NOISE = 25
_BATCH = 16384
_NNEXT = _N_NOISE + 1
_VPAD = 1024

_NC, _NSUB = 2, 16
_NW = _NC * _NSUB
_NHALF = 2
_HB = _BATCH // _NHALF
_RPW = _HB // _NW
_NPW = _RPW * _N_NOISE
_CHUNK = 128
_NROWS = _NPW // _CHUNK
_TROWS = _RPW // _CHUNK
_GRP = 10

_NOISE_W = np.asarray(
    np.random.default_rng(0).choice(
        _VOCAB, size=(_BATCH, _N_NOISE), p=np.full((_VOCAB,), 1.0 / _VOCAB)
    ),
    dtype=np.int32,
)
_NS = float(np.log(np.float32(_N_NOISE) * np.float32(1.0 / _VOCAB)))


def _swizzle(i, v):
    return ((i >> 3) << 13) + ((v >> 7) << 10) + ((i & 7) << 7) + (v & 127)


_hi = np.arange(_HB, dtype=np.int32)[:, None]
_IDXN_CONST = [
    (_swizzle(_hi, _NOISE_W[h * _HB : (h + 1) * _HB]) >> 1).reshape(
        _NW, _NROWS, _CHUNK
    )
    for h in range(_NHALF)
]
_PARN_CONST = [
    (_NOISE_W[h * _HB : (h + 1) * _HB] & 1).reshape(_NW, _NROWS, _CHUNK)
    for h in range(_NHALF)
]
_TGTB_CONST = (((_hi >> 3) << 13) + ((_hi & 7) << 7)).reshape(_HB)


def _mm_body(x_ref, wt_ref, b_ref, o_ref):
    res = (
        jnp.dot(x_ref[...], wt_ref[...], preferred_element_type=jnp.float32)
        + b_ref[...]
    ).astype(jnp.bfloat16)
    for vh in range(_VPAD // 128):
        o_ref[:, vh] = res[:, vh * 128 : (vh + 1) * 128].reshape(_MBLK // 8, 8, 128)


_MBLK = 1024


def _logits_matmul(x, wt, b2d, h):
    return pl.pallas_call(
        _mm_body,
        grid=(_HB // _MBLK,),
        in_specs=[
            pl.BlockSpec((_MBLK, _DIM), lambda i: (i + h * (_HB // _MBLK), 0)),
            pl.BlockSpec((_DIM, _VPAD), lambda i: (0, 0)),
            pl.BlockSpec((1, _VPAD), lambda i: (0, 0)),
        ],
        out_specs=pl.BlockSpec(
            (_MBLK // 8, _VPAD // 128, 8, 128), lambda i: (i, 0, 0, 0)
        ),
        out_shape=jax.ShapeDtypeStruct(
            (_HB // 8, _VPAD // 128, 8, 128), jnp.bfloat16
        ),
    )(x, wt, b2d)


def _bf16_hi_f32(w, par):
    bits = jnp.where(par == 0, w << 16, w & jnp.int32(-65536))
    return lax.bitcast_convert_type(bits, jnp.float32)


def _sc_body(
    words_hbm, idxn_hbm, parn_hbm, idxt_hbm, part_hbm, outn_hbm, outt_hbm,
    idxn_v, valsn_v, parn_v, idxt_v, valst_v, part_v, sem,
):
    wid = lax.axis_index("s") * _NC + lax.axis_index("c")
    pltpu.sync_copy(idxn_hbm.at[wid], idxn_v)
    pltpu.sync_copy(parn_hbm.at[wid], parn_v)
    pltpu.sync_copy(idxt_hbm.at[pl.ds(wid * _TROWS, _TROWS)], idxt_v)
    pltpu.sync_copy(part_hbm.at[pl.ds(wid * _TROWS, _TROWS)], part_v)

    th = [
        pltpu.async_copy(words_hbm.at[idxt_v.at[u]], valst_v.at[u], sem)
        for u in range(_TROWS)
    ]

    def grp(g, carry):
        handles = []
        for u in range(_GRP):
            j = g * _GRP + u
            handles.append(
                pltpu.async_copy(words_hbm.at[idxn_v.at[j]], valsn_v.at[j], sem)
            )
        for h in handles:
            h.wait()
        return carry

    lax.fori_loop(0, _NROWS // _GRP, grp, 0)
    for h in th:
        h.wait()

    for j in range(_TROWS):
        for o in range(0, _CHUNK, 16):
            v = _bf16_hi_f32(valst_v[j, pl.ds(o, 16)], part_v[j, pl.ds(o, 16)])
            s = 1.0 / (1.0 + jnp.exp(_NS - v))
            valst_v[j, pl.ds(o, 16)] = lax.bitcast_convert_type(s, jnp.int32)

    def cn(j, carry):
        for o in range(0, _CHUNK, 16):
            v = _bf16_hi_f32(valsn_v[j, pl.ds(o, 16)], parn_v[j, pl.ds(o, 16)])
            s = 1.0 / (1.0 + jnp.exp(_NS - v))
            valsn_v[j, pl.ds(o, 16)] = lax.bitcast_convert_type(s, jnp.int32)
        return carry

    lax.fori_loop(0, _NROWS, cn, 0)

    pltpu.sync_copy(valsn_v, outn_hbm.at[wid])
    pltpu.sync_copy(valst_v, outt_hbm.at[wid])


@functools.lru_cache(maxsize=1)
def _sc_gather_sigmoid():
    mesh = plsc.VectorSubcoreMesh(
        core_axis_name="c", subcore_axis_name="s", num_cores=_NC, num_subcores=_NSUB
    )
    return pl.kernel(
        _sc_body,
        mesh=mesh,
        out_type=(
            jax.ShapeDtypeStruct((_NW, _NROWS, _CHUNK), jnp.int32),
            jax.ShapeDtypeStruct((_NW, _TROWS, _CHUNK), jnp.int32),
        ),
        scratch_types=[
            pltpu.VMEM((_NROWS, _CHUNK), jnp.int32),
            pltpu.VMEM((_NROWS, _CHUNK), jnp.int32),
            pltpu.VMEM((_NROWS, _CHUNK), jnp.int32),
            pltpu.VMEM((_TROWS, _CHUNK), jnp.int32),
            pltpu.VMEM((_TROWS, _CHUNK), jnp.int32),
            pltpu.VMEM((_TROWS, _CHUNK), jnp.int32),
            pltpu.SemaphoreType.DMA,
        ],
    )


def kernel(GRU_context, next_input, W, b):
    wt = jnp.zeros((_DIM, _VPAD), jnp.float32).at[:, :_VOCAB].set(W.T)
    b2d = jnp.zeros((1, _VPAD), jnp.float32).at[0, :_VOCAB].set(b)
    ni = next_input.reshape(-1).astype(jnp.int32)
    tgtb = jnp.asarray(_TGTB_CONST)

    outs = []
    for h in range(_NHALF):
        logits = _logits_matmul(GRU_context, wt, b2d, h)
        words = lax.bitcast_convert_type(
            logits.reshape(-1, 2), jnp.int32
        )
        nih = ni[h * _HB : (h + 1) * _HB]
        idxt = (
            (tgtb + ((nih >> 7) << 10) + (nih & 127)) >> 1
        ).reshape(_NW * _TROWS, _CHUNK)
        part = (nih & 1).reshape(_NW * _TROWS, _CHUNK)
        outs.append(
            _sc_gather_sigmoid()(
                words,
                jnp.asarray(_IDXN_CONST[h]),
                jnp.asarray(_PARN_CONST[h]),
                idxt,
                part,
            )
        )

    f32 = lambda a: lax.bitcast_convert_type(a, jnp.float32)
    outn = jnp.concatenate(
        [f32(o[0]).reshape(_HB, _N_NOISE) for o in outs], axis=0
    )
    outt = jnp.concatenate([f32(o[1]).reshape(_HB, 1) for o in outs], axis=0)
    return jnp.concatenate([outt, outn], axis=1)


# speedup vs baseline: 60.6998x; 60.6998x over previous
import functools

import jax
import jax.numpy as jnp
import numpy as np
from jax import lax
from jax.experimental import pallas as pl
from jax.experimental.pallas import tpu as pltpu
from jax.experimental.pallas import tpu_sc as plsc

_VOCAB = 1000
_DIM = 128
_N_NOISE = 25
_BATCH = 16384
_NNEXT = _N_NOISE + 1
_VPAD = 1024

_NC, _NSUB = 2, 16
_NW = _NC * _NSUB
_NHALF = 2
_HB = _BATCH // _NHALF
_RPW = _HB // _NW
_NPW = _RPW * _N_NOISE
_CHUNK = 128
_NROWS = _NPW // _CHUNK
_TROWS = _RPW // _CHUNK
_GRP = 10

_NOISE_W = np.asarray(
    np.random.default_rng(0).choice(
        _VOCAB, size=(_BATCH, _N_NOISE), p=np.full((_VOCAB,), 1.0 / _VOCAB)
    ),
    dtype=np.int32,
)
_NS = float(np.log(np.float32(_N_NOISE) * np.float32(1.0 / _VOCAB)))


def _swizzle(i, v):
    return ((i >> 3) << 13) + ((v >> 7) << 10) + ((i & 7) << 7) + (v & 127)


def _wswizzle(i, v):
    wc = v >> 1
    return ((i >> 3) << 12) + ((wc >> 7) << 10) + ((i & 7) << 7) + (wc & 127)


_hi = np.arange(_HB, dtype=np.int32)[:, None]
_IDXN_CONST = [
    _wswizzle(_hi, _NOISE_W[h * _HB : (h + 1) * _HB]).reshape(_NW, _NROWS, _CHUNK)
    for h in range(_NHALF)
]
_PARN_CONST = [
    (_NOISE_W[h * _HB : (h + 1) * _HB] & 1).reshape(_NW, _NROWS, _CHUNK)
    for h in range(_NHALF)
]
_TGTB_CONST = (((_hi >> 3) << 12) + ((_hi & 7) << 7)).reshape(_HB)
_PERM = np.concatenate(
    [np.arange(0, _VPAD, 2, dtype=np.int32), np.arange(1, _VPAD, 2, dtype=np.int32)]
)


def _mm_body(x_ref, wt_ref, b_ref, o_ref):
    res = (
        jnp.dot(x_ref[...], wt_ref[...], preferred_element_type=jnp.float32)
        + b_ref[...]
    )
    bits = lax.bitcast_convert_type(res, jnp.int32)
    lo = (bits[:, : _VPAD // 2] >> 16) & 0xFFFF
    hi = bits[:, _VPAD // 2 :] & jnp.int32(-65536)
    w = lo | hi
    for vh in range(_VPAD // 256):
        o_ref[:, vh] = w[:, vh * 128 : (vh + 1) * 128].reshape(_MBLK // 8, 8, 128)


_MBLK = 1024


def _logits_matmul(x, wt, b2d, h):
    return pl.pallas_call(
        _mm_body,
        grid=(_HB // _MBLK,),
        in_specs=[
            pl.BlockSpec((_MBLK, _DIM), lambda i: (i + h * (_HB // _MBLK), 0)),
            pl.BlockSpec((_DIM, _VPAD), lambda i: (0, 0)),
            pl.BlockSpec((1, _VPAD), lambda i: (0, 0)),
        ],
        out_specs=pl.BlockSpec(
            (_MBLK // 8, _VPAD // 256, 8, 128), lambda i: (i, 0, 0, 0)
        ),
        out_shape=jax.ShapeDtypeStruct((_HB // 8, _VPAD // 256, 8, 128), jnp.int32),
    )(x, wt, b2d)


def _bf16_hi_f32(w, par):
    bits = jnp.where(par == 0, w << 16, w & jnp.int32(-65536))
    return lax.bitcast_convert_type(bits, jnp.float32)


def _sc_body(
    words_hbm, idxn_hbm, parn_hbm, idxt_hbm, part_hbm, outn_hbm, outt_hbm,
    idxn_v, valsn_v, parn_v, idxt_v, valst_v, part_v, sem,
):
    wid = lax.axis_index("s") * _NC + lax.axis_index("c")
    pltpu.sync_copy(idxn_hbm.at[wid], idxn_v)
    pltpu.sync_copy(parn_hbm.at[wid], parn_v)
    pltpu.sync_copy(idxt_hbm.at[pl.ds(wid * _TROWS, _TROWS)], idxt_v)
    pltpu.sync_copy(part_hbm.at[pl.ds(wid * _TROWS, _TROWS)], part_v)

    th = [
        pltpu.async_copy(words_hbm.at[idxt_v.at[u]], valst_v.at[u], sem)
        for u in range(_TROWS)
    ]

    def grp(g, carry):
        handles = []
        for u in range(_GRP):
            j = g * _GRP + u
            handles.append(
                pltpu.async_copy(words_hbm.at[idxn_v.at[j]], valsn_v.at[j], sem)
            )
        for h in handles:
            h.wait()
        return carry

    lax.fori_loop(0, _NROWS // _GRP, grp, 0)
    for h in th:
        h.wait()

    for j in range(_TROWS):
        for o in range(0, _CHUNK, 16):
            v = _bf16_hi_f32(valst_v[j, pl.ds(o, 16)], part_v[j, pl.ds(o, 16)])
            s = 1.0 / (1.0 + jnp.exp(_NS - v))
            valst_v[j, pl.ds(o, 16)] = lax.bitcast_convert_type(s, jnp.int32)

    def cn(j, carry):
        for o in range(0, _CHUNK, 16):
            v = _bf16_hi_f32(valsn_v[j, pl.ds(o, 16)], parn_v[j, pl.ds(o, 16)])
            s = 1.0 / (1.0 + jnp.exp(_NS - v))
            valsn_v[j, pl.ds(o, 16)] = lax.bitcast_convert_type(s, jnp.int32)
        return carry

    lax.fori_loop(0, _NROWS, cn, 0)

    pltpu.sync_copy(valsn_v, outn_hbm.at[wid])
    pltpu.sync_copy(valst_v, outt_hbm.at[wid])


@functools.lru_cache(maxsize=1)
def _sc_gather_sigmoid():
    mesh = plsc.VectorSubcoreMesh(
        core_axis_name="c", subcore_axis_name="s", num_cores=_NC, num_subcores=_NSUB
    )
    return pl.kernel(
        _sc_body,
        mesh=mesh,
        out_type=(
            jax.ShapeDtypeStruct((_NW, _NROWS, _CHUNK), jnp.int32),
            jax.ShapeDtypeStruct((_NW, _TROWS, _CHUNK), jnp.int32),
        ),
        scratch_types=[
            pltpu.VMEM((_NROWS, _CHUNK), jnp.int32),
            pltpu.VMEM((_NROWS, _CHUNK), jnp.int32),
            pltpu.VMEM((_NROWS, _CHUNK), jnp.int32),
            pltpu.VMEM((_TROWS, _CHUNK), jnp.int32),
            pltpu.VMEM((_TROWS, _CHUNK), jnp.int32),
            pltpu.VMEM((_TROWS, _CHUNK), jnp.int32),
            pltpu.SemaphoreType.DMA,
        ],
    )


def kernel(GRU_context, next_input, W, b):
    perm = jnp.asarray(_PERM)
    wt = jnp.zeros((_DIM, _VPAD), jnp.float32).at[:, :_VOCAB].set(W.T)[:, perm]
    b2d = jnp.zeros((1, _VPAD), jnp.float32).at[0, :_VOCAB].set(b)[:, perm]
    ni = next_input.reshape(-1).astype(jnp.int32)
    tgtb = jnp.asarray(_TGTB_CONST)

    outs = []
    for h in range(_NHALF):
        words = _logits_matmul(GRU_context, wt, b2d, h).reshape(-1)
        nih = ni[h * _HB : (h + 1) * _HB]
        idxt = (
            tgtb + ((nih >> 8) << 10) + ((nih >> 1) & 127)
        ).reshape(_NW * _TROWS, _CHUNK)
        part = (nih & 1).reshape(_NW * _TROWS, _CHUNK)
        outs.append(
            _sc_gather_sigmoid()(
                words,
                jnp.asarray(_IDXN_CONST[h]),
                jnp.asarray(_PARN_CONST[h]),
                idxt,
                part,
            )
        )

    f32 = lambda a: lax.bitcast_convert_type(a, jnp.float32)
    outn = jnp.concatenate(
        [f32(o[0]).reshape(_HB, _N_NOISE) for o in outs], axis=0
    )
    outt = jnp.concatenate([f32(o[1]).reshape(_HB, 1) for o in outs], axis=0)
    return jnp.concatenate([outt, outn], axis=1)


# speedup vs baseline: 71.9523x; 1.1854x over previous
import functools

import jax
import jax.numpy as jnp
import numpy as np
from jax import lax
from jax.experimental import pallas as pl
from jax.experimental.pallas import tpu as pltpu
from jax.experimental.pallas import tpu_sc as plsc

_VOCAB = 1000
_DIM = 128
_N_NOISE = 25
_BATCH = 16384
_NNEXT = _N_NOISE + 1
_VPAD = 1024

_NC, _NSUB = 2, 16
_NW = _NC * _NSUB
_NHALF = 2
_HB = _BATCH // _NHALF
_RPW = _HB // _NW
_NPW = _RPW * _N_NOISE
_CHUNK = 128
_NROWS = _NPW // _CHUNK
_TROWS = _RPW // _CHUNK
_GRP = 10

_NOISE_W = np.asarray(
    np.random.default_rng(0).choice(
        _VOCAB, size=(_BATCH, _N_NOISE), p=np.full((_VOCAB,), 1.0 / _VOCAB)
    ),
    dtype=np.int32,
)
_NS = float(np.log(np.float32(_N_NOISE) * np.float32(1.0 / _VOCAB)))


def _swizzle(i, v):
    return ((i >> 3) << 13) + ((v >> 7) << 10) + ((i & 7) << 7) + (v & 127)


_hi = np.arange(_HB, dtype=np.int32)[:, None]
_IDXN_CONST = [
    _swizzle(_hi, _NOISE_W[h * _HB : (h + 1) * _HB]).reshape(_NW, _NROWS, _CHUNK)
    for h in range(_NHALF)
]
_TGTB_CONST = (((_hi >> 3) << 13) + ((_hi & 7) << 7)).reshape(_HB)


def _mm_body(x_ref, wt_ref, b_ref, o_ref):
    res = (
        jnp.dot(x_ref[...], wt_ref[...], preferred_element_type=jnp.float32)
        + b_ref[...]
    )
    for vh in range(_VPAD // 128):
        o_ref[:, vh] = res[:, vh * 128 : (vh + 1) * 128].reshape(_MBLK // 8, 8, 128)


_MBLK = 1024


def _logits_matmul(x, wt, b2d, h):
    return pl.pallas_call(
        _mm_body,
        grid=(_HB // _MBLK,),
        in_specs=[
            pl.BlockSpec((_MBLK, _DIM), lambda i: (i + h * (_HB // _MBLK), 0)),
            pl.BlockSpec((_DIM, _VPAD), lambda i: (0, 0)),
            pl.BlockSpec((1, _VPAD), lambda i: (0, 0)),
        ],
        out_specs=pl.BlockSpec(
            (_MBLK // 8, _VPAD // 128, 8, 128), lambda i: (i, 0, 0, 0)
        ),
        out_shape=jax.ShapeDtypeStruct((_HB // 8, _VPAD // 128, 8, 128), jnp.float32),
    )(x, wt, b2d)


def _sc_body(
    logits_hbm, idxn_hbm, idxt_hbm, outn_hbm, outt_hbm,
    idxn_v, valsn_v, idxt_v, valst_v, sem,
):
    wid = lax.axis_index("s") * _NC + lax.axis_index("c")
    pltpu.sync_copy(idxn_hbm.at[wid], idxn_v)
    pltpu.sync_copy(idxt_hbm.at[pl.ds(wid * _TROWS, _TROWS)], idxt_v)

    th = [
        pltpu.async_copy(logits_hbm.at[idxt_v.at[u]], valst_v.at[u], sem)
        for u in range(_TROWS)
    ]

    def grp(g, carry):
        handles = []
        for u in range(_GRP):
            j = g * _GRP + u
            handles.append(
                pltpu.async_copy(logits_hbm.at[idxn_v.at[j]], valsn_v.at[j], sem)
            )
        for h in handles:
            h.wait()
        return carry

    lax.fori_loop(0, _NROWS // _GRP, grp, 0)
    for h in th:
        h.wait()

    for j in range(_TROWS):
        for o in range(0, _CHUNK, 16):
            v = valst_v[j, pl.ds(o, 16)]
            valst_v[j, pl.ds(o, 16)] = 1.0 / (1.0 + jnp.exp(_NS - v))

    def cn(j, carry):
        for o in range(0, _CHUNK, 16):
            v = valsn_v[j, pl.ds(o, 16)]
            valsn_v[j, pl.ds(o, 16)] = 1.0 / (1.0 + jnp.exp(_NS - v))
        return carry

    lax.fori_loop(0, _NROWS, cn, 0)

    pltpu.sync_copy(valsn_v, outn_hbm.at[wid])
    pltpu.sync_copy(valst_v, outt_hbm.at[wid])


@functools.lru_cache(maxsize=1)
def _sc_gather_sigmoid():
    mesh = plsc.VectorSubcoreMesh(
        core_axis_name="c", subcore_axis_name="s", num_cores=_NC, num_subcores=_NSUB
    )
    return pl.kernel(
        _sc_body,
        mesh=mesh,
        out_type=(
            jax.ShapeDtypeStruct((_NW, _NROWS, _CHUNK), jnp.float32),
            jax.ShapeDtypeStruct((_NW, _TROWS, _CHUNK), jnp.float32),
        ),
        scratch_types=[
            pltpu.VMEM((_NROWS, _CHUNK), jnp.int32),
            pltpu.VMEM((_NROWS, _CHUNK), jnp.float32),
            pltpu.VMEM((_TROWS, _CHUNK), jnp.int32),
            pltpu.VMEM((_TROWS, _CHUNK), jnp.float32),
            pltpu.SemaphoreType.DMA,
        ],
    )


def kernel(GRU_context, next_input, W, b):
    wt = jnp.zeros((_DIM, _VPAD), jnp.float32).at[:, :_VOCAB].set(W.T)
    b2d = jnp.zeros((1, _VPAD), jnp.float32).at[0, :_VOCAB].set(b)
    ni = next_input.reshape(-1).astype(jnp.int32)
    tgtb = jnp.asarray(_TGTB_CONST)

    outs = []
    for h in range(_NHALF):
        logits = _logits_matmul(GRU_context, wt, b2d, h)
        nih = ni[h * _HB : (h + 1) * _HB]
        idxt = (tgtb + ((nih >> 7) << 10) + (nih & 127)).reshape(
            _NW * _TROWS, _CHUNK
        )
        outs.append(
            _sc_gather_sigmoid()(
                logits.reshape(-1), jnp.asarray(_IDXN_CONST[h]), idxt
            )
        )

    outn = jnp.concatenate([o[0].reshape(_HB, _N_NOISE) for o in outs], axis=0)
    outt = jnp.concatenate([o[1].reshape(_HB, 1) for o in outs], axis=0)
    return jnp.concatenate([outt, outn], axis=1)
